# baseline (speedup 1.0000x reference)
import functools

import jax
import jax.numpy as jnp
from jax import lax
from jax.experimental import pallas as pl
from jax.experimental.pallas import tpu as pltpu
from jax.experimental.pallas import tpu_sc as plsc

BLK = 128


@functools.lru_cache(maxsize=None)
def _make_gather(hist, n_bt, d, n_workers):
    n_chunks = hist * n_bt
    cpw = n_chunks // n_workers
    dt, di = d // 8, 8
    assert cpw % 2 == 0 and cpw >= 4
    mesh = plsc.VectorSubcoreMesh(core_axis_name="c", subcore_axis_name="s")
    info = plsc.get_sparse_core_info()
    nc = info.num_cores

    @functools.partial(
        pl.kernel,
        mesh=mesh,
        out_type=jax.ShapeDtypeStruct((hist, dt, n_bt, di, BLK), jnp.float32),
        scratch_types=[
            pltpu.VMEM((2, BLK), jnp.int32),
            pltpu.VMEM((2, BLK, d), jnp.float32),
            pltpu.VMEM((2, dt, di, BLK), jnp.float32),
            pltpu.SemaphoreType.DMA((2,)),
            pltpu.SemaphoreType.DMA((2,)),
            pltpu.SemaphoreType.DMA((2,)),
        ],
        compiler_params=pltpu.CompilerParams(
            use_tc_tiling_on_sc=False, needs_layout_passes=False
        ),
    )
    def k(idx_hbm, table_hbm, out_hbm, idx_v, rows_v, t_v, isem, gsem, osem):
        wid = lax.axis_index("s") * nc + lax.axis_index("c")
        base = wid * cpw
        last = n_chunks - 1
        iota = lax.iota(jnp.int32, 16)

        def idx_dma(c, s):
            c = jnp.minimum(c, last)
            h, bt = c // n_bt, c % n_bt
            pltpu.async_copy(idx_hbm.at[h, pl.ds(bt * BLK, BLK)],
                             idx_v.at[s], isem.at[s])

        def wait_isem(s):
            pltpu.make_async_copy(idx_hbm.at[0, pl.ds(0, BLK)],
                                  idx_v.at[s], isem.at[s]).wait()

        def gather(s):
            pltpu.async_copy(table_hbm.at[idx_v.at[s]], rows_v.at[s],
                             gsem.at[s])

        def wait_gsem(s):
            pltpu.make_async_copy(table_hbm.at[pl.ds(0, BLK)],
                                  rows_v.at[s], gsem.at[s]).wait()

        def wait_osem(s):
            pltpu.make_async_copy(t_v.at[s], out_hbm.at[0, :, 0],
                                  osem.at[s]).wait()

        def transpose(s):
            def col(dd, carry):
                cvec = jnp.full((16,), dd, jnp.int32)
                for bg in range(8):
                    rvec = iota + (bg * 16)
                    v = plsc.load_gather(rows_v.at[s], [rvec, cvec])
                    t_v[s, dd // di, dd % di, pl.ds(bg * 16, 16)] = v
                return carry
            lax.fori_loop(0, d, col, 0)

        def out_dma(c, s):
            h, bt = c // n_bt, c % n_bt
            pltpu.async_copy(t_v.at[s], out_hbm.at[h, :, bt], osem.at[s])

        idx_dma(base, 0)
        idx_dma(base + 1, 1)
        wait_isem(0)
        gather(0)

        def body(jj, carry):
            for s in (0, 1):
                o = 1 - s
                j = jj * 2 + s
                g = base + j
                wait_isem(o)
                gather(o)
                wait_gsem(s)
                idx_dma(g + 2, s)

                @pl.when(j >= 2)
                def _():
                    wait_osem(s)

                transpose(s)
                out_dma(g, s)
            return carry

        lax.fori_loop(0, cpw // 2, body, 0)

        wait_gsem(cpw % 2)
        wait_isem((cpw + 1) % 2)
        wait_osem(0)
        wait_osem(1)

    return k


def kernel(input_text, table):
    b, h = input_text.shape
    v, d = table.shape
    n_bt = b // BLK
    idx_t = input_text.T.astype(jnp.int32)
    out5 = _make_gather(h, n_bt, d, 32)(idx_t, table)
    return out5.transpose(2, 4, 0, 1, 3).reshape(b, h, d)
